# Initial kernel scaffold; baseline (speedup 1.0000x reference)
import jax
import jax.numpy as jnp
from jax.experimental import pallas as pl


def kernel(x, codebooks):
    raise NotImplementedError("write your pallas kernel here")


# speedup vs baseline: 1.1736x; 1.1736x over previous
import functools

import jax
import jax.numpy as jnp
from jax.experimental import pallas as pl

NUM_CODEBOOKS = 4
CODEBOOK_SIZE = 1024
EMBED_DIM = 256
BATCH = 16384

TILE_B = 1024


def _level_body(x_ref, res_ref, r2_ref, recon_ref, cb_ref, c2_ref,
                res_out_ref, recon_out_ref, code_ref, loss_ref):
    res = res_ref[...]
    cb = cb_ref[0]
    r2 = r2_ref[...][:, :1]
    c2 = c2_ref[...][:1, :]

    dot = jax.lax.dot_general(
        res.astype(jnp.bfloat16), cb.astype(jnp.bfloat16),
        (((1,), (1,)), ((), ())),
        preferred_element_type=jnp.float32)
    dist = (r2 + c2) - 2.0 * dot
    mn = jnp.min(dist, axis=-1, keepdims=True)
    iota = jax.lax.broadcasted_iota(jnp.int32, dist.shape, 1)
    code = jnp.min(jnp.where(dist == mn, iota, CODEBOOK_SIZE), axis=-1)

    onehot = (jax.lax.broadcasted_iota(jnp.int32, (TILE_B, CODEBOOK_SIZE), 1)
              == code[:, None]).astype(jnp.bfloat16)
    c_hi = cb.astype(jnp.bfloat16)
    rem = cb - c_hi.astype(jnp.float32)
    c_mid = rem.astype(jnp.bfloat16)
    c_lo = (rem - c_mid.astype(jnp.float32)).astype(jnp.bfloat16)
    dg = lambda a, b: jax.lax.dot_general(
        a, b, (((1,), (0,)), ((), ())), preferred_element_type=jnp.float32)
    q = (dg(onehot, c_hi) + dg(onehot, c_mid)) + dg(onehot, c_lo)

    diff = q - res
    recon = recon_ref[...] + q
    res_new = x_ref[...] - recon

    res_out_ref[...] = res_new
    recon_out_ref[...] = recon
    code_ref[...] = code[:, None] + jnp.zeros((TILE_B, 8), jnp.int32)
    @pl.when(pl.program_id(0) == 0)
    def _init():
        loss_ref[...] = jnp.zeros_like(loss_ref)
    loss_ref[...] += jnp.sum(diff * diff)[None, None]


def _level_call(x, res, r2, recon, cb_i, c2_i):
    num_tiles = BATCH // TILE_B
    return pl.pallas_call(
        _level_body,
        grid=(num_tiles,),
        in_specs=[
            pl.BlockSpec((TILE_B, EMBED_DIM), lambda b: (b, 0)),
            pl.BlockSpec((TILE_B, EMBED_DIM), lambda b: (b, 0)),
            pl.BlockSpec((TILE_B, 8), lambda b: (b, 0)),
            pl.BlockSpec((TILE_B, EMBED_DIM), lambda b: (b, 0)),
            pl.BlockSpec((1, CODEBOOK_SIZE, EMBED_DIM), lambda b: (0, 0, 0)),
            pl.BlockSpec((8, CODEBOOK_SIZE), lambda b: (0, 0)),
        ],
        out_specs=[
            pl.BlockSpec((TILE_B, EMBED_DIM), lambda b: (b, 0)),
            pl.BlockSpec((TILE_B, EMBED_DIM), lambda b: (b, 0)),
            pl.BlockSpec((TILE_B, 8), lambda b: (b, 0)),
            pl.BlockSpec((1, 1), lambda b: (0, 0)),
        ],
        out_shape=[
            jax.ShapeDtypeStruct((BATCH, EMBED_DIM), jnp.float32),
            jax.ShapeDtypeStruct((BATCH, EMBED_DIM), jnp.float32),
            jax.ShapeDtypeStruct((BATCH, 8), jnp.int32),
            jax.ShapeDtypeStruct((1, 1), jnp.float32),
        ],
    )(x, res, r2, recon, cb_i, c2_i)


@jax.jit
def kernel(x, codebooks):
    residual = x
    recon = jnp.zeros_like(x)
    codes = []
    loss = jnp.zeros((), jnp.float32)
    for i in range(NUM_CODEBOOKS):
        r2 = jnp.sum(residual ** 2, axis=-1, keepdims=True)
        r2b = jnp.broadcast_to(r2, (BATCH, 8))
        c2b = jnp.broadcast_to(c2[i][None, :], (8, CODEBOOK_SIZE))
        residual, recon, code, loss_part = _level_call(
            x, residual, r2b, recon, codebooks[i:i + 1], c2b)
        codes.append(code[:, 0])
        loss = loss + loss_part[0, 0]
    codes_tensor = jnp.stack(codes, axis=-1)
    total_loss = (loss * 2.0) / (BATCH * EMBED_DIM)
    return recon, codes_tensor, total_loss


# speedup vs baseline: 1.4595x; 1.2436x over previous
import jax
import jax.numpy as jnp
from jax.experimental import pallas as pl

NUM_CODEBOOKS = 4
CODEBOOK_SIZE = 1024
EMBED_DIM = 256
BATCH = 16384

TILE_B = 1024


def _rqvae_body(x_ref, cb_ref, recon_ref, codes_ref, loss_ref):
    x = x_ref[...]
    recon = jnp.zeros_like(x)
    residual = x
    loss_part = jnp.zeros((), dtype=jnp.float32)
    codes_list = []
    for i in range(NUM_CODEBOOKS):
        cb = cb_ref[i]
        c2 = jnp.sum(cb * cb, axis=-1)
        r2 = jnp.sum(residual * residual, axis=-1, keepdims=True)
        dot = jax.lax.dot_general(
            residual.astype(jnp.bfloat16), cb.astype(jnp.bfloat16),
            (((1,), (1,)), ((), ())),
            preferred_element_type=jnp.float32)
        dist = (r2 + c2[None, :]) - 2.0 * dot
        mn = jnp.min(dist, axis=-1, keepdims=True)
        iota = jax.lax.broadcasted_iota(jnp.int32, dist.shape, 1)
        code = jnp.min(jnp.where(dist == mn, iota, CODEBOOK_SIZE), axis=-1)

        onehot = (jax.lax.broadcasted_iota(jnp.int32, dist.shape, 1)
                  == code[:, None]).astype(jnp.bfloat16)
        c_hi = cb.astype(jnp.bfloat16)
        rem = cb - c_hi.astype(jnp.float32)
        c_mid = rem.astype(jnp.bfloat16)
        c_lo = (rem - c_mid.astype(jnp.float32)).astype(jnp.bfloat16)
        dg = lambda a, b: jax.lax.dot_general(
            a, b, (((1,), (0,)), ((), ())), preferred_element_type=jnp.float32)
        q = (dg(onehot, c_hi) + dg(onehot, c_mid)) + dg(onehot, c_lo)

        diff = q - residual
        loss_part = loss_part + jnp.sum(diff * diff)
        recon = recon + q
        residual = x - recon
        codes_list.append(code)

    recon_ref[...] = recon
    codes_ref[...] = jnp.stack(codes_list, axis=-1)
    @pl.when(pl.program_id(0) == 0)
    def _init():
        loss_ref[...] = jnp.zeros_like(loss_ref)
    loss_ref[...] += loss_part[None, None]


@jax.jit
def kernel(x, codebooks):
    num_tiles = BATCH // TILE_B
    recon, codes, loss_sum = pl.pallas_call(
        _rqvae_body,
        grid=(num_tiles,),
        in_specs=[
            pl.BlockSpec((TILE_B, EMBED_DIM), lambda b: (b, 0)),
            pl.BlockSpec((NUM_CODEBOOKS, CODEBOOK_SIZE, EMBED_DIM),
                         lambda b: (0, 0, 0)),
        ],
        out_specs=[
            pl.BlockSpec((TILE_B, EMBED_DIM), lambda b: (b, 0)),
            pl.BlockSpec((TILE_B, NUM_CODEBOOKS), lambda b: (b, 0)),
            pl.BlockSpec((1, 1), lambda b: (0, 0)),
        ],
        out_shape=[
            jax.ShapeDtypeStruct((BATCH, EMBED_DIM), jnp.float32),
            jax.ShapeDtypeStruct((BATCH, NUM_CODEBOOKS), jnp.int32),
            jax.ShapeDtypeStruct((1, 1), jnp.float32),
        ],
    )(x, codebooks)
    total_loss = (loss_sum[0, 0] * 2.0) / (BATCH * EMBED_DIM)
    return recon, codes, total_loss
